# baseline (speedup 1.0000x reference)
import functools

import jax
import jax.numpy as jnp
import numpy as np
from jax import lax
from jax.experimental import pallas as pl
from jax.experimental.pallas import tpu as pltpu
from jax.experimental.pallas import tpu_sc as plsc

NBC = 16
NROWS = 1600
NCOLS = 1000
HALF_ROWS = 800
CHUNK_ROWS = 40
NCHUNKS = 63
RW = 1024
RHALF = 24 * RW
HALF_OUT = 10240


def _col0(k):
    return 16 * k if k < 62 else 984


_GROUPS = ((0, 16), (16, 32), (32, 48), (48, NCHUNKS))

_mesh = plsc.VectorSubcoreMesh(core_axis_name="c", subcore_axis_name="s")


def _worker():
    c_idx = lax.axis_index("c")
    s_idx = lax.axis_index("s")
    pair = s_idx // 2
    j = s_idx % 2
    bc = c_idx * 8 + pair
    return bc, j


@functools.partial(
    pl.kernel,
    mesh=_mesh,
    out_type=jax.ShapeDtypeStruct((NBC * 2 * RHALF,), jnp.float32),
    scratch_types=[
        pltpu.VMEM((CHUNK_ROWS, NCOLS), jnp.float32),
        pltpu.VMEM((CHUNK_ROWS, NCOLS), jnp.float32),
        pltpu.VMEM((RHALF,), jnp.float32),
        pltpu.SemaphoreType.DMA,
        pltpu.SemaphoreType.DMA,
    ],
)
def _reduce_sc(x_hbm, r_hbm, in0, in1, r_my, sem0, sem1):
    bc, j = _worker()
    row_base = j * HALF_ROWS

    def dma(blk, buf, sem):
        return pltpu.make_async_copy(
            x_hbm.at[bc, pl.ds(row_base + blk * CHUNK_ROWS, CHUNK_ROWS), :],
            buf, sem)

    def accumulate(blk, buf):
        for g0, g1 in _GROUPS:
            nk = g1 - g0

            def r_body(r2, acc, _g0=g0, _nk=nk):
                r = 2 * r2
                acc = tuple(
                    acc[i] + buf[r, pl.ds(_col0(_g0 + i), 16)]
                    for i in range(_nk))
                return tuple(
                    acc[i] + buf[r + 1, pl.ds(_col0(_g0 + i), 16)]
                    for i in range(_nk))

            acc = lax.fori_loop(
                0, CHUNK_ROWS // 2, r_body,
                tuple(jnp.zeros((16,), jnp.float32) for _ in range(nk)))
            for i in range(nk):
                r_my[pl.ds(blk * RW + _col0(g0 + i), 16)] = acc[i]

    dma(0, in0, sem0).start()
    dma(1, in1, sem1).start()

    def m_body(m, carry):
        dma(2 * m, in0, sem0).wait()
        accumulate(2 * m, in0)

        @pl.when(m < 9)
        def _():
            dma(2 * m + 2, in0, sem0).start()

        dma(2 * m + 1, in1, sem1).wait()
        accumulate(2 * m + 1, in1)

        @pl.when(m < 9)
        def _():
            dma(2 * m + 3, in1, sem1).start()

        return carry

    lax.fori_loop(0, 10, m_body, 0)

    pltpu.sync_copy(r_my, r_hbm.at[pl.ds((bc * 2 + j) * RHALF, RHALF)])


def _stitch_tables():
    t = np.arange(2 * HALF_OUT)
    s = t // 20
    u = t % 20
    idx1 = (u * RW + np.minimum(s, 999)).astype(np.int32)
    msk1 = np.where(s <= 999, 0.025, 0.0).astype(np.float32)
    idx2 = (u * RW + np.clip(s - 1, 0, 999)).astype(np.int32)
    msk2 = np.where((s >= 1) & (s <= 1000), 0.025, 0.0).astype(np.float32)
    return idx1, msk1, idx2, msk2


_IDX1, _MSK1, _IDX2, _MSK2 = _stitch_tables()


@functools.partial(
    pl.kernel,
    mesh=_mesh,
    compiler_params=pltpu.CompilerParams(
        use_tc_tiling_on_sc=False, needs_layout_passes=False),
    out_type=(
        jax.ShapeDtypeStruct((NBC, 8, 1280), jnp.float32),
        jax.ShapeDtypeStruct((NBC, 8, 1280), jnp.float32),
    ),
    scratch_types=[
        pltpu.VMEM((20 * RW,), jnp.float32),
        pltpu.VMEM((20 * RW,), jnp.float32),
        pltpu.VMEM((HALF_OUT,), jnp.int32),
        pltpu.VMEM((HALF_OUT,), jnp.float32),
        pltpu.VMEM((HALF_OUT,), jnp.int32),
        pltpu.VMEM((HALF_OUT,), jnp.float32),
        pltpu.VMEM((8, 1280), jnp.float32),
    ],
)
def _stitch_sc(r_hbm, idx1_hbm, msk1_hbm, idx2_hbm, msk2_hbm,
               out_a_hbm, out_b_hbm, r_a, r_b, i1, m1, i2, m2, out_v):
    bc, j = _worker()
    t0 = j * HALF_OUT

    pltpu.sync_copy(r_hbm.at[pl.ds((bc * 2) * RHALF, 20 * RW)], r_a)
    pltpu.sync_copy(r_hbm.at[pl.ds((bc * 2 + 1) * RHALF, 20 * RW)], r_b)
    pltpu.sync_copy(idx1_hbm.at[pl.ds(t0, HALF_OUT)], i1)
    pltpu.sync_copy(msk1_hbm.at[pl.ds(t0, HALF_OUT)], m1)
    pltpu.sync_copy(idx2_hbm.at[pl.ds(t0, HALF_OUT)], i2)
    pltpu.sync_copy(msk2_hbm.at[pl.ds(t0, HALF_OUT)], m2)

    def rr_body(rr, carry):
        def ii_body(ii, carry2):
            o = rr * 1280 + 16 * ii
            g1 = plsc.load_gather(r_a, [i1[pl.ds(o, 16)]])
            g2 = plsc.load_gather(r_b, [i2[pl.ds(o, 16)]])
            out_v[rr, pl.ds(16 * ii, 16)] = (
                g1 * m1[pl.ds(o, 16)] + g2 * m2[pl.ds(o, 16)])
            return carry2

        lax.fori_loop(0, 80, ii_body, 0)
        return carry

    lax.fori_loop(0, 8, rr_body, 0)

    @pl.when(j == 0)
    def _():
        pltpu.sync_copy(out_v, out_a_hbm.at[bc])

    @pl.when(j == 1)
    def _():
        pltpu.sync_copy(out_v, out_b_hbm.at[bc])


@jax.jit
def kernel(est_source):
    x = est_source.reshape(NBC, NROWS, NCOLS)
    r = _reduce_sc(x)
    out_a, out_b = _stitch_sc(
        r, jnp.asarray(_IDX1), jnp.asarray(_MSK1),
        jnp.asarray(_IDX2), jnp.asarray(_MSK2))
    full = jnp.concatenate(
        [out_a.reshape(NBC, HALF_OUT), out_b.reshape(NBC, HALF_OUT)], axis=1)
    return full[:, :20020].reshape(8, 2, 20020)


# speedup vs baseline: 1.1059x; 1.1059x over previous
import functools

import jax
import jax.numpy as jnp
import numpy as np
from jax import lax
from jax.experimental import pallas as pl
from jax.experimental.pallas import tpu as pltpu
from jax.experimental.pallas import tpu_sc as plsc

NBC = 16
NROWS = 1600
NCOLS = 1000
HALF_ROWS = 800
CHUNK_ROWS = 40
NCHUNKS = 63
RW = 1024
RHALF = 24 * RW
HALF_OUT = 10240


def _col0(k):
    return 16 * k if k < 62 else 984


_GROUPS = ((0, 16), (16, 32), (32, 48), (48, NCHUNKS))

_mesh = plsc.VectorSubcoreMesh(core_axis_name="c", subcore_axis_name="s")


def _worker():
    c_idx = lax.axis_index("c")
    s_idx = lax.axis_index("s")
    pair = s_idx // 2
    j = s_idx % 2
    bc = c_idx * 8 + pair
    return bc, j


@functools.partial(
    pl.kernel,
    mesh=_mesh,
    out_type=jax.ShapeDtypeStruct((NBC * 2 * RHALF,), jnp.float32),
    scratch_types=[
        pltpu.VMEM((CHUNK_ROWS, NCOLS), jnp.float32),
        pltpu.VMEM((CHUNK_ROWS, NCOLS), jnp.float32),
        pltpu.VMEM((RHALF,), jnp.float32),
        pltpu.SemaphoreType.DMA,
        pltpu.SemaphoreType.DMA,
    ],
)
def _reduce_sc(x_hbm, r_hbm, in0, in1, r_my, sem0, sem1):
    bc, j = _worker()
    row_base = j * HALF_ROWS

    def dma(blk, buf, sem):
        return pltpu.make_async_copy(
            x_hbm.at[bc, pl.ds(row_base + blk * CHUNK_ROWS, CHUNK_ROWS), :],
            buf, sem)

    def accumulate(blk, buf):
        for g0, g1 in _GROUPS:
            nk = g1 - g0

            def r_body(r2, acc, _g0=g0, _nk=nk):
                r = 2 * r2
                acc = tuple(
                    acc[i] + buf[r, pl.ds(_col0(_g0 + i), 16)]
                    for i in range(_nk))
                return tuple(
                    acc[i] + buf[r + 1, pl.ds(_col0(_g0 + i), 16)]
                    for i in range(_nk))

            acc = lax.fori_loop(
                0, CHUNK_ROWS // 2, r_body,
                tuple(jnp.zeros((16,), jnp.float32) for _ in range(nk)))
            for i in range(nk):
                r_my[pl.ds(blk * RW + _col0(g0 + i), 16)] = acc[i]

    dma(0, in0, sem0).start()
    dma(1, in1, sem1).start()

    def m_body(m, carry):
        dma(2 * m, in0, sem0).wait()
        accumulate(2 * m, in0)

        @pl.when(m < 9)
        def _():
            dma(2 * m + 2, in0, sem0).start()

        dma(2 * m + 1, in1, sem1).wait()
        accumulate(2 * m + 1, in1)

        @pl.when(m < 9)
        def _():
            dma(2 * m + 3, in1, sem1).start()

        return carry

    lax.fori_loop(0, 10, m_body, 0)

    pltpu.sync_copy(r_my, r_hbm.at[pl.ds((bc * 2 + j) * RHALF, RHALF)])


@functools.partial(
    pl.kernel,
    mesh=_mesh,
    compiler_params=pltpu.CompilerParams(
        use_tc_tiling_on_sc=False, needs_layout_passes=False),
    out_type=(
        jax.ShapeDtypeStruct((NBC, HALF_OUT), jnp.float32),
        jax.ShapeDtypeStruct((NBC, HALF_OUT), jnp.float32),
    ),
    scratch_types=[
        pltpu.VMEM((20 * RW,), jnp.float32),
        pltpu.VMEM((20 * RW,), jnp.float32),
        pltpu.VMEM((HALF_OUT,), jnp.float32),
    ],
)
def _stitch_sc(r_hbm, out_a_hbm, out_b_hbm, r_a, r_b, out_v):
    bc, j = _worker()
    t0 = j * HALF_OUT

    pltpu.sync_copy(r_hbm.at[pl.ds((bc * 2) * RHALF, 20 * RW)], r_a)
    pltpu.sync_copy(r_hbm.at[pl.ds((bc * 2 + 1) * RHALF, 20 * RW)], r_b)

    zero = jnp.zeros((16,), jnp.float32)
    iot = lax.iota(jnp.int32, 16)
    scale = jnp.float32(0.025)

    def sample_idx(i):
        t = t0 + 16 * i + iot
        tf = t.astype(jnp.float32)
        s = (tf * jnp.float32(0.05) + jnp.float32(1e-3)).astype(jnp.int32)
        u = t - 20 * s
        return s, u

    def masked_chunk(i):
        s, u = sample_idx(i)
        u_row = lax.shift_left(u, 10)
        g1 = plsc.load_gather(r_a, [u_row + jnp.minimum(s, 999)])
        v1 = jnp.where(s <= 999, g1, zero)
        col2 = jnp.minimum(jnp.maximum(s - 1, 0), 999)
        g2 = plsc.load_gather(r_b, [u_row + col2])
        v2 = jnp.where((s >= 1) & (s <= 1000), g2, zero)
        out_v[pl.ds(16 * i, 16)] = (v1 + v2) * scale

    def interior_chunk(i):
        s, u = sample_idx(i)
        idx1 = lax.shift_left(u, 10) + s
        g1 = plsc.load_gather(r_a, [idx1])
        g2 = plsc.load_gather(r_b, [idx1 - 1])
        out_v[pl.ds(16 * i, 16)] = (g1 + g2) * scale

    def interior_loop(lo, hi):
        def body(k, carry):
            interior_chunk(2 * k)
            interior_chunk(2 * k + 1)
            return carry

        lax.fori_loop(lo // 2, hi // 2, body, 0)

    @pl.when(j == 0)
    def _():
        masked_chunk(0)
        masked_chunk(1)
        interior_loop(2, 640)
        pltpu.sync_copy(out_v, out_a_hbm.at[bc])

    @pl.when(j == 1)
    def _():
        interior_loop(0, 610)
        masked_chunk(610)
        masked_chunk(611)

        def zbody(i, carry):
            out_v[pl.ds(16 * i, 16)] = zero
            return carry

        lax.fori_loop(612, 640, zbody, 0)
        pltpu.sync_copy(out_v, out_b_hbm.at[bc])


@jax.jit
def kernel(est_source):
    x = est_source.reshape(NBC, NROWS, NCOLS)
    r = _reduce_sc(x)
    out_a, out_b = _stitch_sc(r)
    full = jnp.concatenate([out_a, out_b], axis=1)
    return full[:, :20020].reshape(8, 2, 20020)


# speedup vs baseline: 1.1713x; 1.0591x over previous
import functools

import jax
import jax.numpy as jnp
import numpy as np
from jax import lax
from jax.experimental import pallas as pl
from jax.experimental.pallas import tpu as pltpu
from jax.experimental.pallas import tpu_sc as plsc

NBC = 16
NROWS = 1600
NCOLS = 1000
HALF_ROWS = 800
CHUNK_ROWS = 40
NCHUNKS = 63
RW = 1024
RHALF = 24 * RW
HALF_OUT = 10240


def _col0(k):
    return 16 * k if k < 62 else 984


_GROUPS = ((0, 16), (16, 32), (32, 48), (48, NCHUNKS))

_mesh = plsc.VectorSubcoreMesh(core_axis_name="c", subcore_axis_name="s")


def _worker():
    c_idx = lax.axis_index("c")
    s_idx = lax.axis_index("s")
    pair = s_idx // 2
    j = s_idx % 2
    bc = c_idx * 8 + pair
    return bc, j


@functools.partial(
    pl.kernel,
    mesh=_mesh,
    out_type=jax.ShapeDtypeStruct((NBC * 2 * RHALF,), jnp.float32),
    scratch_types=[
        pltpu.VMEM((CHUNK_ROWS, NCOLS), jnp.float32),
        pltpu.VMEM((CHUNK_ROWS, NCOLS), jnp.float32),
        pltpu.VMEM((RHALF,), jnp.float32),
        pltpu.SemaphoreType.DMA,
        pltpu.SemaphoreType.DMA,
    ],
)
def _reduce_sc(x_hbm, r_hbm, in0, in1, r_my, sem0, sem1):
    bc, j = _worker()
    row_base = j * HALF_ROWS

    def dma(blk, buf, sem):
        return pltpu.make_async_copy(
            x_hbm.at[bc, pl.ds(row_base + blk * CHUNK_ROWS, CHUNK_ROWS), :],
            buf, sem)

    def accumulate(blk, buf):
        for g0, g1 in _GROUPS:
            nk = g1 - g0
            zeros = tuple(jnp.zeros((16,), jnp.float32) for _ in range(nk))

            def r_body(r, acc, _g0=g0, _nk=nk):
                return tuple(
                    acc[i] + buf[r, pl.ds(_col0(_g0 + i), 16)]
                    for i in range(_nk))

            acc = plsc.parallel_loop(
                0, CHUNK_ROWS, unroll=4, carry=zeros)(r_body)
            for i in range(nk):
                r_my[pl.ds(blk * RW + _col0(g0 + i), 16)] = acc[i]

    dma(0, in0, sem0).start()
    dma(1, in1, sem1).start()

    def m_body(m, carry):
        dma(2 * m, in0, sem0).wait()
        accumulate(2 * m, in0)

        @pl.when(m < 9)
        def _():
            dma(2 * m + 2, in0, sem0).start()

        dma(2 * m + 1, in1, sem1).wait()
        accumulate(2 * m + 1, in1)

        @pl.when(m < 9)
        def _():
            dma(2 * m + 3, in1, sem1).start()

        return carry

    lax.fori_loop(0, 10, m_body, 0)

    pltpu.sync_copy(r_my, r_hbm.at[pl.ds((bc * 2 + j) * RHALF, RHALF)])


@functools.partial(
    pl.kernel,
    mesh=_mesh,
    compiler_params=pltpu.CompilerParams(
        use_tc_tiling_on_sc=False, needs_layout_passes=False),
    out_type=(
        jax.ShapeDtypeStruct((NBC, HALF_OUT), jnp.float32),
        jax.ShapeDtypeStruct((NBC, HALF_OUT), jnp.float32),
    ),
    scratch_types=[
        pltpu.VMEM((20 * RW,), jnp.float32),
        pltpu.VMEM((20 * RW,), jnp.float32),
        pltpu.VMEM((HALF_OUT,), jnp.float32),
    ],
)
def _stitch_sc(r_hbm, out_a_hbm, out_b_hbm, r_a, r_b, out_v):
    bc, j = _worker()
    t0 = j * HALF_OUT

    pltpu.sync_copy(r_hbm.at[pl.ds((bc * 2) * RHALF, 20 * RW)], r_a)
    pltpu.sync_copy(r_hbm.at[pl.ds((bc * 2 + 1) * RHALF, 20 * RW)], r_b)

    zero = jnp.zeros((16,), jnp.float32)
    iot = lax.iota(jnp.int32, 16)
    scale = jnp.float32(0.025)

    def sample_idx(i):
        t = t0 + 16 * i + iot
        tf = t.astype(jnp.float32)
        s = (tf * jnp.float32(0.05) + jnp.float32(1e-3)).astype(jnp.int32)
        u = t - 20 * s
        return s, u

    def masked_chunk(i):
        s, u = sample_idx(i)
        u_row = lax.shift_left(u, 10)
        g1 = plsc.load_gather(r_a, [u_row + jnp.minimum(s, 999)])
        v1 = jnp.where(s <= 999, g1, zero)
        col2 = jnp.minimum(jnp.maximum(s - 1, 0), 999)
        g2 = plsc.load_gather(r_b, [u_row + col2])
        v2 = jnp.where((s >= 1) & (s <= 1000), g2, zero)
        out_v[pl.ds(16 * i, 16)] = (v1 + v2) * scale

    def interior_chunk(i):
        s, u = sample_idx(i)
        idx1 = lax.shift_left(u, 10) + s
        g1 = plsc.load_gather(r_a, [idx1])
        g2 = plsc.load_gather(r_b, [idx1 - 1])
        out_v[pl.ds(16 * i, 16)] = (g1 + g2) * scale

    def interior_loop(lo, hi):
        plsc.parallel_loop(lo, hi, unroll=4)(interior_chunk)

    @pl.when(j == 0)
    def _():
        masked_chunk(0)
        masked_chunk(1)
        interior_loop(2, 640)
        pltpu.sync_copy(out_v, out_a_hbm.at[bc])

    @pl.when(j == 1)
    def _():
        interior_loop(0, 610)
        masked_chunk(610)
        masked_chunk(611)

        def zbody(i, carry):
            out_v[pl.ds(16 * i, 16)] = zero
            return carry

        lax.fori_loop(612, 640, zbody, 0)
        pltpu.sync_copy(out_v, out_b_hbm.at[bc])


@jax.jit
def kernel(est_source):
    x = est_source.reshape(NBC, NROWS, NCOLS)
    r = _reduce_sc(x)
    out_a, out_b = _stitch_sc(r)
    full = jnp.concatenate([out_a, out_b], axis=1)
    return full[:, :20020].reshape(8, 2, 20020)


# speedup vs baseline: 1.1821x; 1.0092x over previous
import functools

import jax
import jax.numpy as jnp
import numpy as np
from jax import lax
from jax.experimental import pallas as pl
from jax.experimental.pallas import tpu as pltpu
from jax.experimental.pallas import tpu_sc as plsc

NBC = 16
NROWS = 1600
NCOLS = 1000
HALF_ROWS = 800
CHUNK_ROWS = 40
NCHUNKS = 63
RW = 1024
RHALF = 24 * RW
HALF_OUT = 10240


def _col0(k):
    return 16 * k if k < 62 else 984


_GROUPS = ((0, 16), (16, 32), (32, 48), (48, NCHUNKS))

_mesh = plsc.VectorSubcoreMesh(core_axis_name="c", subcore_axis_name="s")


def _worker():
    c_idx = lax.axis_index("c")
    s_idx = lax.axis_index("s")
    pair = s_idx // 2
    j = s_idx % 2
    bc = c_idx * 8 + pair
    return bc, j


@functools.partial(
    pl.kernel,
    mesh=_mesh,
    out_type=jax.ShapeDtypeStruct((NBC * 2 * RHALF,), jnp.float32),
    scratch_types=[
        pltpu.VMEM((CHUNK_ROWS, NCOLS), jnp.float32),
        pltpu.VMEM((CHUNK_ROWS, NCOLS), jnp.float32),
        pltpu.VMEM((RHALF,), jnp.float32),
        pltpu.SemaphoreType.DMA,
        pltpu.SemaphoreType.DMA,
    ],
)
def _reduce_sc(x_hbm, r_hbm, in0, in1, r_my, sem0, sem1):
    bc, j = _worker()
    row_base = j * HALF_ROWS

    def dma(blk, buf, sem):
        return pltpu.make_async_copy(
            x_hbm.at[bc, pl.ds(row_base + blk * CHUNK_ROWS, CHUNK_ROWS), :],
            buf, sem)

    def accumulate(blk, buf):
        for g0, g1 in _GROUPS:
            nk = g1 - g0
            zeros = tuple(jnp.zeros((16,), jnp.float32) for _ in range(nk))

            def r_body(r, acc, _g0=g0, _nk=nk):
                return tuple(
                    acc[i] + buf[r, pl.ds(_col0(_g0 + i), 16)]
                    for i in range(_nk))

            acc = plsc.parallel_loop(
                0, CHUNK_ROWS, unroll=4, carry=zeros)(r_body)
            for i in range(nk):
                r_my[pl.ds(blk * RW + _col0(g0 + i), 16)] = acc[i]

    dma(0, in0, sem0).start()
    dma(1, in1, sem1).start()

    def m_body(m, carry):
        dma(2 * m, in0, sem0).wait()
        accumulate(2 * m, in0)

        @pl.when(m < 9)
        def _():
            dma(2 * m + 2, in0, sem0).start()

        dma(2 * m + 1, in1, sem1).wait()
        accumulate(2 * m + 1, in1)

        @pl.when(m < 9)
        def _():
            dma(2 * m + 3, in1, sem1).start()

        return carry

    lax.fori_loop(0, 10, m_body, 0)

    pltpu.sync_copy(r_my, r_hbm.at[pl.ds((bc * 2 + j) * RHALF, RHALF)])


@functools.partial(
    pl.kernel,
    mesh=_mesh,
    compiler_params=pltpu.CompilerParams(
        use_tc_tiling_on_sc=False, needs_layout_passes=False),
    out_type=jax.ShapeDtypeStruct((NBC, 2 * HALF_OUT), jnp.float32),
    scratch_types=[
        pltpu.VMEM((20 * RW,), jnp.float32),
        pltpu.VMEM((20 * RW,), jnp.float32),
        pltpu.VMEM((HALF_OUT,), jnp.float32),
        pltpu.SemaphoreType.DMA,
        pltpu.SemaphoreType.DMA,
    ],
)
def _stitch_sc(r_hbm, out_hbm, r_a, r_b, out_v, sem_a, sem_b):
    bc, j = _worker()
    t0 = j * HALF_OUT

    cp_a = pltpu.make_async_copy(
        r_hbm.at[pl.ds((bc * 2) * RHALF, 20 * RW)], r_a, sem_a)
    cp_b = pltpu.make_async_copy(
        r_hbm.at[pl.ds((bc * 2 + 1) * RHALF, 20 * RW)], r_b, sem_b)
    cp_a.start()
    cp_b.start()
    cp_a.wait()
    cp_b.wait()

    zero = jnp.zeros((16,), jnp.float32)
    iot = lax.iota(jnp.int32, 16)
    scale = jnp.float32(0.025)

    def sample_idx(i):
        t = t0 + 16 * i + iot
        tf = t.astype(jnp.float32)
        s = (tf * jnp.float32(0.05) + jnp.float32(1e-3)).astype(jnp.int32)
        u = t - 20 * s
        return s, u

    def masked_chunk(i):
        s, u = sample_idx(i)
        u_row = lax.shift_left(u, 10)
        g1 = plsc.load_gather(r_a, [u_row + jnp.minimum(s, 999)])
        v1 = jnp.where(s <= 999, g1, zero)
        col2 = jnp.minimum(jnp.maximum(s - 1, 0), 999)
        g2 = plsc.load_gather(r_b, [u_row + col2])
        v2 = jnp.where((s >= 1) & (s <= 1000), g2, zero)
        out_v[pl.ds(16 * i, 16)] = (v1 + v2) * scale

    def interior_chunk(i):
        s, u = sample_idx(i)
        idx1 = lax.shift_left(u, 10) + s
        g1 = plsc.load_gather(r_a, [idx1])
        g2 = plsc.load_gather(r_b, [idx1 - 1])
        out_v[pl.ds(16 * i, 16)] = (g1 + g2) * scale

    def interior_loop(lo, hi):
        plsc.parallel_loop(lo, hi, unroll=8)(interior_chunk)

    @pl.when(j == 0)
    def _():
        masked_chunk(0)
        masked_chunk(1)
        interior_loop(2, 640)

    @pl.when(j == 1)
    def _():
        interior_loop(0, 610)
        masked_chunk(610)
        masked_chunk(611)

        def zbody(i):
            out_v[pl.ds(16 * i, 16)] = zero

        plsc.parallel_loop(612, 640)(zbody)

    pltpu.sync_copy(out_v, out_hbm.at[bc, pl.ds(t0, HALF_OUT)])


@jax.jit
def kernel(est_source):
    x = est_source.reshape(NBC, NROWS, NCOLS)
    r = _reduce_sc(x)
    full = _stitch_sc(r)
    return full[:, :20020].reshape(8, 2, 20020)
